# baseline (speedup 1.0000x reference)
import functools

import jax
import jax.numpy as jnp
from jax import lax
from jax.experimental import pallas as pl
from jax.experimental.pallas import tpu as pltpu
from jax.experimental.pallas import tpu_sc as plsc

_NUM_CORES = 2
_NUM_SUBCORES = 16
_NW = _NUM_CORES * _NUM_SUBCORES
_LANES = 16
_NBUF = 4
_DEPTH = 2
_SLOT = 128


def _emb_kernel(B, S, D, V):
    rows_w = B * S // _NW
    n_slots = rows_w // _SLOT
    trips = n_slots // _NBUF
    tail = n_slots - trips * _NBUF
    ipw = rows_w // _SLOT
    mesh = plsc.VectorSubcoreMesh(
        core_axis_name="c", subcore_axis_name="s",
        num_cores=_NUM_CORES, num_subcores=_NUM_SUBCORES)

    @functools.partial(
        pl.kernel,
        out_type=jax.ShapeDtypeStruct((B * S, D), jnp.float32),
        mesh=mesh,
        scratch_types=[
            pltpu.VMEM((ipw + 6, _SLOT), jnp.int32),
            pltpu.VMEM((S + _SLOT - 8, D), jnp.float32),
            [pltpu.VMEM((_SLOT, D), jnp.float32) for _ in range(_NBUF)],
            [pltpu.SemaphoreType.DMA for _ in range(_NBUF)],
            [pltpu.SemaphoreType.DMA for _ in range(_NBUF)],
            pltpu.SemaphoreType.DMA,
        ],
    )
    def body(x_hbm, pos_hbm, tok_hbm, out_hbm,
             idx_v, pos_v, bufs, gs, os, ps):
        wid = lax.axis_index("s") * _NUM_CORES + lax.axis_index("c")
        row0 = wid * rows_w
        d = lax.rem(wid * ipw, 8)
        lo = pl.multiple_of(wid * ipw - d, 8)

        pltpu.sync_copy(x_hbm.at[pl.ds(lo, ipw + 6)], idx_v)
        pos_cp = pltpu.async_copy(pos_hbm, pos_v, ps)

        def gather(j, k):
            pltpu.async_copy(tok_hbm.at[idx_v.at[d + j]], bufs[k], gs[k])

        def gather_wait(k):
            pltpu.make_async_copy(tok_hbm.at[idx_v.at[0]], bufs[k],
                                  gs[k]).wait()

        def out_issue(j, k):
            pltpu.async_copy(bufs[k],
                             out_hbm.at[pl.ds(row0 + j * _SLOT, _SLOT)],
                             os[k])

        def out_wait(k):
            pltpu.make_async_copy(bufs[k], out_hbm.at[pl.ds(0, _SLOT)],
                                  os[k]).wait()

        def add_pos(j, k):
            off = lax.rem(j * _SLOT, S)

            def add_row(r, c):
                for ch in range(D // _LANES):
                    sl = pl.ds(ch * _LANES, _LANES)
                    plsc.addupdate(bufs[k].at[r, sl], pos_v[off + r, sl])
                return c
            lax.fori_loop(0, _SLOT, add_row, 0, unroll=2)

        for k in range(_DEPTH):
            gather(k, k)
        pos_cp.wait()

        def step(g, carry):
            for k in range(_NBUF):
                j = _NBUF * g + k
                gather_wait(k)
                kn = (k + _DEPTH) % _NBUF
                if k < _DEPTH:
                    @pl.when(g >= 1)
                    def _():
                        out_wait(kn)
                else:
                    out_wait(kn)
                gather(j + _DEPTH, kn)
                add_pos(j, k)
                out_issue(j, k)
            return carry

        lax.fori_loop(0, trips, step, 0, unroll=False)

        for k in range(tail):
            j = trips * _NBUF + k
            gather_wait(k)
            out_wait((k + _DEPTH) % _NBUF)
            if j + _DEPTH < n_slots:
                gather(j + _DEPTH, (k + _DEPTH) % _NBUF)
            add_pos(j, k)
            out_issue(j, k)
        for k in range(tail - _DEPTH, tail):
            out_wait(k % _NBUF)

    return body


def kernel(x, pos_table, token_table):
    B, S = x.shape
    V, D = token_table.shape
    x2 = x.astype(jnp.int32).reshape(B * S // _SLOT, _SLOT)
    pos_ext = jnp.concatenate([pos_table, pos_table[: _SLOT - 8]], axis=0)
    out = _emb_kernel(B, S, D, V)(x2, pos_ext, token_table)
    return out.reshape(B, S, D)


# speedup vs baseline: 2.2825x; 2.2825x over previous
import functools

import jax
import jax.numpy as jnp
from jax import lax
from jax.experimental import pallas as pl
from jax.experimental.pallas import tpu as pltpu
from jax.experimental.pallas import tpu_sc as plsc

_NUM_CORES = 2
_NUM_SUBCORES = 16
_NW = _NUM_CORES * _NUM_SUBCORES
_LANES = 16
_NBUF = 4
_DEPTH = 2
_SA = 104
_SB = 96


def _emb_kernel(B, S, D, V):
    b_per_w = B // _NW
    n_slots = 2 * b_per_w
    trips = n_slots // _NBUF
    tail = n_slots - trips * _NBUF
    mesh = plsc.VectorSubcoreMesh(
        core_axis_name="c", subcore_axis_name="s",
        num_cores=_NUM_CORES, num_subcores=_NUM_SUBCORES)
    slot_rows = (_SA, _SB)

    @functools.partial(
        pl.kernel,
        out_type=jax.ShapeDtypeStruct((B, S, D), jnp.float32),
        mesh=mesh,
        scratch_types=[
            pltpu.VMEM((b_per_w, _SA), jnp.int32),
            pltpu.VMEM((b_per_w, _SB), jnp.int32),
            pltpu.VMEM((S, D), jnp.float32),
            [pltpu.VMEM((_SA, D), jnp.float32) for _ in range(_NBUF)],
            [pltpu.SemaphoreType.DMA for _ in range(_NBUF)],
            [pltpu.SemaphoreType.DMA for _ in range(_NBUF)],
            pltpu.SemaphoreType.DMA,
        ],
    )
    def body(xa_hbm, xb_hbm, pos_hbm, tok_hbm, out_hbm,
             idx_a, idx_b, pos_v, bufs, gs, os, ps):
        wid = lax.axis_index("s") * _NUM_CORES + lax.axis_index("c")
        base = wid * b_per_w
        idxs = (idx_a, idx_b)

        pltpu.sync_copy(xa_hbm.at[pl.ds(base, b_per_w)], idx_a)
        pltpu.sync_copy(xb_hbm.at[pl.ds(base, b_per_w)], idx_b)
        pos_cp = pltpu.async_copy(pos_hbm, pos_v, ps)

        def gather(bat, k):
            h = k % 2
            pltpu.async_copy(tok_hbm.at[idxs[h].at[bat]],
                             bufs[k].at[pl.ds(0, slot_rows[h])], gs[k])

        def gather_wait(k):
            h = k % 2
            pltpu.make_async_copy(tok_hbm.at[idxs[h].at[0]],
                                  bufs[k].at[pl.ds(0, slot_rows[h])],
                                  gs[k]).wait()

        def out_issue(bat, k):
            h = k % 2
            pltpu.async_copy(
                bufs[k].at[pl.ds(0, slot_rows[h])],
                out_hbm.at[base + bat, pl.ds(h * _SA, slot_rows[h])], os[k])

        def out_wait(k):
            h = k % 2
            pltpu.make_async_copy(
                bufs[k].at[pl.ds(0, slot_rows[h])],
                out_hbm.at[base, pl.ds(h * _SA, slot_rows[h])], os[k]).wait()

        def add_pos(k):
            h = k % 2

            def add_row(r, c):
                for ch in range(D // _LANES):
                    sl = pl.ds(ch * _LANES, _LANES)
                    plsc.addupdate(bufs[k].at[r, sl],
                                   pos_v[h * _SA + r, sl])
                return c
            lax.fori_loop(0, slot_rows[h], add_row, 0, unroll=2)

        for k in range(_DEPTH):
            gather(k // 2, k)
        pos_cp.wait()

        def step(g, carry):
            for k in range(_NBUF):
                bat = (_NBUF // 2) * g + k // 2
                gather_wait(k)
                kn = (k + _DEPTH) % _NBUF
                if k < _DEPTH:
                    @pl.when(g >= 1)
                    def _():
                        out_wait(kn)
                else:
                    out_wait(kn)
                if k + _DEPTH < _NBUF:
                    gather((_NBUF // 2) * g + (k + _DEPTH) // 2, kn)
                else:
                    @pl.when(_NBUF * g + k + _DEPTH < n_slots)
                    def _():
                        gather((_NBUF // 2) * g + (k + _DEPTH) // 2, kn)
                add_pos(k)
                out_issue(bat, k)
            return carry

        lax.fori_loop(0, trips, step, 0, unroll=False)

        for k in range(tail):
            i = trips * _NBUF + k
            gather_wait(k)
            out_wait((k + _DEPTH) % _NBUF)
            if i + _DEPTH < n_slots:
                gather((i + _DEPTH) // 2, (k + _DEPTH) % _NBUF)
            add_pos(k)
            out_issue(i // 2, k)
        for k in range(tail - _DEPTH, tail):
            out_wait(k % _NBUF)

    return body


def kernel(x, pos_table, token_table):
    B, S = x.shape
    V, D = token_table.shape
    xi = x.astype(jnp.int32)
    out = _emb_kernel(B, S, D, V)(
        xi[:, :_SA], xi[:, _SA:], pos_table, token_table)
    return out
